# baseline (speedup 1.0000x reference)
import functools

import jax
import jax.numpy as jnp
from jax import lax
from jax.experimental import pallas as pl
from jax.experimental.pallas import tpu as pltpu
from jax.experimental.pallas import tpu_sc as plsc

NUM_GRAPHS = 64
_D = 128

_NC = 2
_NS = 16
_NW = _NC * _NS


def _make_edge_kernel(n_nodes, d, rows_pad, rp):
    rp0, rp1 = rp
    assert 16 * (rp0 + rp1) == rows_pad and rp0 % 8 == 0 and rp1 % 8 == 0
    units = n_nodes // 8
    assert n_nodes % 8 == 0
    upt = units // _NS
    urem = units % _NS
    rpt = upt * 8
    zr = max(z for z in range(8, 129, 8) if rpt % z == 0)
    nz = rpt // zr
    acc_rows = n_nodes + 16
    ch = 24

    mesh = plsc.VectorSubcoreMesh(core_axis_name="c", subcore_axis_name="s")

    @functools.partial(
        pl.kernel,
        mesh=mesh,
        out_type=jax.ShapeDtypeStruct((_NC, n_nodes, d), jnp.float32),
        scratch_types=[
            pltpu.VMEM((ch, 128), jnp.int32),
            pltpu.VMEM((ch, 128), jnp.int32),
            pltpu.VMEM((128, d), jnp.float32),
            pltpu.VMEM((128, d), jnp.float32),
            pltpu.VMEM_SHARED((acc_rows, d), jnp.float32),
            pltpu.SemaphoreType.DMA,
            pltpu.SemaphoreType.DMA,
        ],
    )
    def edge_kernel(m_hbm, src_hbm, dst_hbm, out_hbm,
                    srcv, dstv, rows0, rows1, acc, sem0, sem1):
        c = lax.axis_index("c")
        s = lax.axis_index("s")

        def zbody(i, _):
            r = i // (d // 16)
            k = (i % (d // 16)) * 16
            rows0[r, pl.ds(k, 16)] = jnp.zeros((16,), jnp.float32)
            return 0

        with jax.named_scope("ph1_zero"):
            lax.fori_loop(0, zr * (d // 16), zbody, 0)
            for j in range(nz):
                pltpu.sync_copy(rows0.at[pl.ds(0, zr)],
                                acc.at[pl.ds(s * rpt + j * zr, zr)])
            if urem:
                @pl.when(s < urem)
                def _zx():
                    pltpu.sync_copy(rows0.at[pl.ds(0, 8)],
                                    acc.at[pl.ds(_NS * rpt + s * 8, 8)])
            plsc.subcore_barrier()

        nrows = jnp.where(c == 0, rp0, rp1)
        row0 = jnp.where(c == 0, s * rp0, 16 * rp0 + s * rp1)

        def pair_body(cnt):
            def pair(p, _):
                j = 2 * p
                pltpu.async_copy(m_hbm.at[srcv.at[j + 1]], rows1, sem1)
                pltpu.make_async_copy(
                    m_hbm.at[srcv.at[j]], rows0, sem0).wait()
                pltpu.sync_copy(rows0, acc.at[dstv.at[j]], add=True)

                @pl.when(j + 2 < cnt)
                def _nxt():
                    pltpu.async_copy(m_hbm.at[srcv.at[j + 2]], rows0, sem0)

                pltpu.make_async_copy(
                    m_hbm.at[srcv.at[j + 1]], rows1, sem1).wait()
                pltpu.sync_copy(rows1, acc.at[dstv.at[j + 1]], add=True)
                return 0
            return pair

        def chunk_body(k, _):
            base = k * ch
            cnt = jnp.minimum(ch, nrows - base)
            pltpu.sync_copy(src_hbm.at[pl.ds(row0 + base, ch)], srcv)
            pltpu.sync_copy(dst_hbm.at[pl.ds(row0 + base, ch)], dstv)
            pltpu.async_copy(m_hbm.at[srcv.at[0]], rows0, sem0)
            lax.fori_loop(0, cnt // 2, pair_body(cnt), 0)
            return 0

        with jax.named_scope("ph2_edges"):
            lax.fori_loop(0, (nrows + ch - 1) // ch, chunk_body, 0)
            plsc.subcore_barrier()

        with jax.named_scope("ph3_dump"):
            pltpu.sync_copy(acc.at[pl.ds(s * rpt, rpt)],
                            out_hbm.at[c].at[pl.ds(s * rpt, rpt)])
            if urem:
                @pl.when(s < urem)
                def _dx():
                    o = _NS * rpt + s * 8
                    pltpu.sync_copy(acc.at[pl.ds(o, 8)],
                                    out_hbm.at[c].at[pl.ds(o, 8)])

    return edge_kernel


_CORE_SPLIT = (160, 0)


def _edge_agg(m, src2d, dst2d):
    n, d = m.shape
    rows_total = src2d.shape[0]
    unit = _NW * 8
    rows_pad = ((rows_total + unit - 1) // unit) * unit
    pad = rows_pad - rows_total + 24
    src2d = jnp.concatenate(
        [src2d, jnp.zeros((pad, 128), jnp.int32)], axis=0)
    dst2d = jnp.concatenate(
        [dst2d, jnp.full((pad, 128), n, jnp.int32)], axis=0)
    return _make_edge_kernel(n, d, rows_pad, _CORE_SPLIT)(m, src2d, dst2d)


def _mm_body(x_ref, w_ref, o_ref):
    o_ref[:] = jnp.dot(x_ref[:], w_ref[:], preferred_element_type=jnp.float32)


def _matmul(x, w):
    n, d = x.shape
    r = 1000
    return pl.pallas_call(
        _mm_body,
        grid=(n // r,),
        in_specs=[pl.BlockSpec((r, d), lambda i: (i, 0)),
                  pl.BlockSpec((d, d), lambda i: (0, 0))],
        out_specs=pl.BlockSpec((r, d), lambda i: (i, 0)),
        out_shape=jax.ShapeDtypeStruct((n, d), jnp.float32),
    )(x, w)


def _gru_body(p0, p1, h, wih, whh, bih, bhh, hout):
    d = _D
    agg = p0[:] + p1[:]
    gi = jnp.dot(agg, wih[:], preferred_element_type=jnp.float32) + bih[:]
    gh = jnp.dot(h[:], whh[:], preferred_element_type=jnp.float32) + bhh[:]
    r = jax.nn.sigmoid(gi[:, :d] + gh[:, :d])
    z = jax.nn.sigmoid(gi[:, d:2 * d] + gh[:, d:2 * d])
    n = jnp.tanh(gi[:, 2 * d:] + r * gh[:, 2 * d:])
    hout[:] = (1.0 - z) * n + z * h[:]


def _gru_fused_body(p0, p1, h, wih, whh, bih, bhh, wl, hout, mout):
    d = _D
    agg = p0[:] + p1[:]
    gi = jnp.dot(agg, wih[:], preferred_element_type=jnp.float32) + bih[:]
    gh = jnp.dot(h[:], whh[:], preferred_element_type=jnp.float32) + bhh[:]
    r = jax.nn.sigmoid(gi[:, :d] + gh[:, :d])
    z = jax.nn.sigmoid(gi[:, d:2 * d] + gh[:, d:2 * d])
    n = jnp.tanh(gi[:, 2 * d:] + r * gh[:, 2 * d:])
    hn = (1.0 - z) * n + z * h[:]
    hout[:] = hn
    mout[:] = jnp.dot(hn, wl[:], preferred_element_type=jnp.float32)


def _gru(parts, h, wiht, whht, bih2, bhh2, wl_next=None):
    n, d = h.shape
    r = 1000
    p0, p1 = parts[0], parts[1]
    row = lambda i: (i, 0)
    full = lambda i: (0, 0)
    in_specs = [
        pl.BlockSpec((r, d), row),
        pl.BlockSpec((r, d), row),
        pl.BlockSpec((r, d), row),
        pl.BlockSpec((d, 3 * d), full),
        pl.BlockSpec((d, 3 * d), full),
        pl.BlockSpec((1, 3 * d), full),
        pl.BlockSpec((1, 3 * d), full),
    ]
    if wl_next is None:
        return pl.pallas_call(
            _gru_body,
            grid=(n // r,),
            in_specs=in_specs,
            out_specs=pl.BlockSpec((r, d), row),
            out_shape=jax.ShapeDtypeStruct((n, d), jnp.float32),
        )(p0, p1, h, wiht, whht, bih2, bhh2)
    return pl.pallas_call(
        _gru_fused_body,
        grid=(n // r,),
        in_specs=in_specs + [pl.BlockSpec((d, d), full)],
        out_specs=[pl.BlockSpec((r, d), row), pl.BlockSpec((r, d), row)],
        out_shape=[jax.ShapeDtypeStruct((n, d), jnp.float32),
                   jax.ShapeDtypeStruct((n, d), jnp.float32)],
    )(p0, p1, h, wiht, whht, bih2, bhh2, wl_next)


def _pool_body(h_ref, x_ref, b_ref, wg_ref, bg_ref, o_ref):
    d = _D
    g = NUM_GRAPHS
    n = h_ref.shape[0]
    h = h_ref[:]
    x = x_ref[:]
    wg = wg_ref[:]
    gate = (jnp.sum(h * wg[:, :d], axis=1, keepdims=True)
            + jnp.sum(x * wg[:, d:], axis=1, keepdims=True)
            + bg_ref[0, 0])
    seg = lax.broadcasted_iota(jnp.int32, (n, g), 1)
    mask = b_ref[:] == seg
    maskf = mask.astype(jnp.float32)
    gm = jnp.max(jnp.where(mask, gate, -1e30), axis=0, keepdims=True)
    gmax_n = jnp.sum(maskf * gm, axis=1, keepdims=True)
    e = jnp.exp(gate - gmax_n)
    den = jnp.sum(maskf * e, axis=0, keepdims=True)
    den_n = jnp.sum(maskf * den, axis=1, keepdims=True)
    a = e / (den_n + 1e-16)
    attn = maskf * a
    dn = (((0,), (0,)), ((), ()))
    o_ref[:, :d] = lax.dot_general(attn, h, dn,
                                   preferred_element_type=jnp.float32)
    o_ref[:, d:] = lax.dot_general(attn, x, dn,
                                   preferred_element_type=jnp.float32)


def _pool(h, x, batch2d, wg2d, bg2d):
    n, d = h.shape
    full = lambda: (0, 0)
    return pl.pallas_call(
        _pool_body,
        grid=(1,),
        in_specs=[
            pl.BlockSpec((n, d), lambda i: (0, 0)),
            pl.BlockSpec((n, d), lambda i: (0, 0)),
            pl.BlockSpec((n, 1), lambda i: (0, 0)),
            pl.BlockSpec((1, 2 * d), lambda i: (0, 0)),
            pl.BlockSpec((1, 1), lambda i: (0, 0)),
        ],
        out_specs=pl.BlockSpec((NUM_GRAPHS, 2 * d), lambda i: (0, 0)),
        out_shape=jax.ShapeDtypeStruct((NUM_GRAPHS, 2 * d), jnp.float32),
    )(h, x, batch2d, wg2d, bg2d)


def kernel(x, edge_index, batch, Wl, W_ih, W_hh, b_ih, b_hh, Wg, bg):
    n, d = x.shape
    num_layers = Wl.shape[0]

    src2d = edge_index[0].reshape(-1, 128)
    dst2d = edge_index[1].reshape(-1, 128)
    wiht = W_ih.T
    whht = W_hh.T
    bih2 = b_ih.reshape(1, -1)
    bhh2 = b_hh.reshape(1, -1)
    batch2d = batch.reshape(n, 1)
    wg2d = Wg.reshape(1, -1)
    bg2d = bg.reshape(1, 1)

    h = x
    m = _matmul(x, Wl[0])
    for i in range(num_layers):
        parts = _edge_agg(m, src2d, dst2d)
        if i + 1 < num_layers:
            h, m = _gru(parts, h, wiht, whht, bih2, bhh2, Wl[i + 1])
        else:
            h = _gru(parts, h, wiht, whht, bih2, bhh2)
    return _pool(h, x, batch2d, wg2d, bg2d)


# speedup vs baseline: 1.1957x; 1.1957x over previous
import functools

import jax
import jax.numpy as jnp
from jax import lax
from jax.experimental import pallas as pl
from jax.experimental.pallas import tpu as pltpu
from jax.experimental.pallas import tpu_sc as plsc

NUM_GRAPHS = 64
_D = 128

_NC = 2
_NS = 16
_NW = _NC * _NS


def _make_edge_kernel(n_nodes, d, rows_pad, rp):
    rp0, rp1 = rp
    assert 16 * (rp0 + rp1) == rows_pad and rp0 % 8 == 0 and rp1 % 8 == 0
    units = n_nodes // 8
    assert n_nodes % 8 == 0
    upt = units // _NS
    urem = units % _NS
    rpt = upt * 8
    zr = max(z for z in range(8, 129, 8) if rpt % z == 0)
    nz = rpt // zr
    acc_rows = n_nodes + 16
    ch = 24

    mesh = plsc.VectorSubcoreMesh(core_axis_name="c", subcore_axis_name="s")

    @functools.partial(
        pl.kernel,
        mesh=mesh,
        out_type=jax.ShapeDtypeStruct((_NC, n_nodes, d), jnp.float32),
        scratch_types=[
            pltpu.VMEM((ch, 128), jnp.int32),
            pltpu.VMEM((ch, 128), jnp.int32),
            pltpu.VMEM((128, d), jnp.float32),
            pltpu.VMEM((128, d), jnp.float32),
            pltpu.VMEM_SHARED((acc_rows, d), jnp.float32),
            pltpu.SemaphoreType.DMA,
            pltpu.SemaphoreType.DMA,
        ],
    )
    def edge_kernel(m_hbm, src_hbm, dst_hbm, out_hbm,
                    srcv, dstv, rows0, rows1, acc, sem0, sem1):
        c = lax.axis_index("c")
        s = lax.axis_index("s")

        def zbody(i, _):
            r = i // (d // 16)
            k = (i % (d // 16)) * 16
            rows0[r, pl.ds(k, 16)] = jnp.zeros((16,), jnp.float32)
            return 0

        with jax.named_scope("ph1_zero"):
            lax.fori_loop(0, zr * (d // 16), zbody, 0)
            for j in range(nz):
                pltpu.sync_copy(rows0.at[pl.ds(0, zr)],
                                acc.at[pl.ds(s * rpt + j * zr, zr)])
            if urem:
                @pl.when(s < urem)
                def _zx():
                    pltpu.sync_copy(rows0.at[pl.ds(0, 8)],
                                    acc.at[pl.ds(_NS * rpt + s * 8, 8)])
            plsc.subcore_barrier()

        nrows = jnp.where(c == 0, rp0, rp1)
        row0 = jnp.where(c == 0, s * rp0, 16 * rp0 + s * rp1)

        def pair_body(cnt):
            def pair(p, _):
                j = 2 * p
                pltpu.async_copy(m_hbm.at[srcv.at[j + 1]], rows1, sem1)
                pltpu.make_async_copy(
                    m_hbm.at[srcv.at[j]], rows0, sem0).wait()
                pltpu.sync_copy(rows0, acc.at[dstv.at[j]], add=True)

                @pl.when(j + 2 < cnt)
                def _nxt():
                    pltpu.async_copy(m_hbm.at[srcv.at[j + 2]], rows0, sem0)

                pltpu.make_async_copy(
                    m_hbm.at[srcv.at[j + 1]], rows1, sem1).wait()
                pltpu.sync_copy(rows1, acc.at[dstv.at[j + 1]], add=True)
                return 0
            return pair

        def chunk_body(k, _):
            base = k * ch
            cnt = jnp.minimum(ch, nrows - base)
            pltpu.sync_copy(src_hbm.at[pl.ds(row0 + base, ch)], srcv)
            pltpu.sync_copy(dst_hbm.at[pl.ds(row0 + base, ch)], dstv)
            pltpu.async_copy(m_hbm.at[srcv.at[0]], rows0, sem0)
            lax.fori_loop(0, cnt // 2, pair_body(cnt), 0)
            return 0

        with jax.named_scope("ph2_edges"):
            lax.fori_loop(0, (nrows + ch - 1) // ch, chunk_body, 0)
            plsc.subcore_barrier()

        with jax.named_scope("ph3_dump"):
            pltpu.sync_copy(acc.at[pl.ds(s * rpt, rpt)],
                            out_hbm.at[c].at[pl.ds(s * rpt, rpt)])
            if urem:
                @pl.when(s < urem)
                def _dx():
                    o = _NS * rpt + s * 8
                    pltpu.sync_copy(acc.at[pl.ds(o, 8)],
                                    out_hbm.at[c].at[pl.ds(o, 8)])

    return edge_kernel


_CORE_SPLIT = (136, 24)


def _edge_agg(m, src2d, dst2d):
    n, d = m.shape
    rows_total = src2d.shape[0]
    unit = _NW * 8
    rows_pad = ((rows_total + unit - 1) // unit) * unit
    pad = rows_pad - rows_total + 24
    src2d = jnp.concatenate(
        [src2d, jnp.zeros((pad, 128), jnp.int32)], axis=0)
    dst2d = jnp.concatenate(
        [dst2d, jnp.full((pad, 128), n, jnp.int32)], axis=0)
    return _make_edge_kernel(n, d, rows_pad, _CORE_SPLIT)(m, src2d, dst2d)


def _mm_body(x_ref, w_ref, o_ref):
    o_ref[:] = jnp.dot(x_ref[:], w_ref[:], preferred_element_type=jnp.float32)


def _matmul(x, w):
    n, d = x.shape
    r = 1000
    return pl.pallas_call(
        _mm_body,
        grid=(n // r,),
        in_specs=[pl.BlockSpec((r, d), lambda i: (i, 0)),
                  pl.BlockSpec((d, d), lambda i: (0, 0))],
        out_specs=pl.BlockSpec((r, d), lambda i: (i, 0)),
        out_shape=jax.ShapeDtypeStruct((n, d), jnp.float32),
    )(x, w)


def _gru_body(p0, p1, h, wih, whh, bih, bhh, hout):
    d = _D
    agg = p0[:] + p1[:]
    gi = jnp.dot(agg, wih[:], preferred_element_type=jnp.float32) + bih[:]
    gh = jnp.dot(h[:], whh[:], preferred_element_type=jnp.float32) + bhh[:]
    r = jax.nn.sigmoid(gi[:, :d] + gh[:, :d])
    z = jax.nn.sigmoid(gi[:, d:2 * d] + gh[:, d:2 * d])
    n = jnp.tanh(gi[:, 2 * d:] + r * gh[:, 2 * d:])
    hout[:] = (1.0 - z) * n + z * h[:]


def _gru_fused_body(p0, p1, h, wih, whh, bih, bhh, wl, hout, mout):
    d = _D
    agg = p0[:] + p1[:]
    gi = jnp.dot(agg, wih[:], preferred_element_type=jnp.float32) + bih[:]
    gh = jnp.dot(h[:], whh[:], preferred_element_type=jnp.float32) + bhh[:]
    r = jax.nn.sigmoid(gi[:, :d] + gh[:, :d])
    z = jax.nn.sigmoid(gi[:, d:2 * d] + gh[:, d:2 * d])
    n = jnp.tanh(gi[:, 2 * d:] + r * gh[:, 2 * d:])
    hn = (1.0 - z) * n + z * h[:]
    hout[:] = hn
    mout[:] = jnp.dot(hn, wl[:], preferred_element_type=jnp.float32)


def _gru(parts, h, wiht, whht, bih2, bhh2, wl_next=None):
    n, d = h.shape
    r = 1000
    p0, p1 = parts[0], parts[1]
    row = lambda i: (i, 0)
    full = lambda i: (0, 0)
    in_specs = [
        pl.BlockSpec((r, d), row),
        pl.BlockSpec((r, d), row),
        pl.BlockSpec((r, d), row),
        pl.BlockSpec((d, 3 * d), full),
        pl.BlockSpec((d, 3 * d), full),
        pl.BlockSpec((1, 3 * d), full),
        pl.BlockSpec((1, 3 * d), full),
    ]
    if wl_next is None:
        return pl.pallas_call(
            _gru_body,
            grid=(n // r,),
            in_specs=in_specs,
            out_specs=pl.BlockSpec((r, d), row),
            out_shape=jax.ShapeDtypeStruct((n, d), jnp.float32),
        )(p0, p1, h, wiht, whht, bih2, bhh2)
    return pl.pallas_call(
        _gru_fused_body,
        grid=(n // r,),
        in_specs=in_specs + [pl.BlockSpec((d, d), full)],
        out_specs=[pl.BlockSpec((r, d), row), pl.BlockSpec((r, d), row)],
        out_shape=[jax.ShapeDtypeStruct((n, d), jnp.float32),
                   jax.ShapeDtypeStruct((n, d), jnp.float32)],
    )(p0, p1, h, wiht, whht, bih2, bhh2, wl_next)


def _pool_body(h_ref, x_ref, b_ref, wg_ref, bg_ref, o_ref):
    d = _D
    g = NUM_GRAPHS
    n = h_ref.shape[0]
    h = h_ref[:]
    x = x_ref[:]
    wg = wg_ref[:]
    gate = (jnp.sum(h * wg[:, :d], axis=1, keepdims=True)
            + jnp.sum(x * wg[:, d:], axis=1, keepdims=True)
            + bg_ref[0, 0])
    seg = lax.broadcasted_iota(jnp.int32, (n, g), 1)
    mask = b_ref[:] == seg
    maskf = mask.astype(jnp.float32)
    gm = jnp.max(jnp.where(mask, gate, -1e30), axis=0, keepdims=True)
    gmax_n = jnp.sum(maskf * gm, axis=1, keepdims=True)
    e = jnp.exp(gate - gmax_n)
    den = jnp.sum(maskf * e, axis=0, keepdims=True)
    den_n = jnp.sum(maskf * den, axis=1, keepdims=True)
    a = e / (den_n + 1e-16)
    attn = maskf * a
    dn = (((0,), (0,)), ((), ()))
    o_ref[:, :d] = lax.dot_general(attn, h, dn,
                                   preferred_element_type=jnp.float32)
    o_ref[:, d:] = lax.dot_general(attn, x, dn,
                                   preferred_element_type=jnp.float32)


def _pool(h, x, batch2d, wg2d, bg2d):
    n, d = h.shape
    full = lambda: (0, 0)
    return pl.pallas_call(
        _pool_body,
        grid=(1,),
        in_specs=[
            pl.BlockSpec((n, d), lambda i: (0, 0)),
            pl.BlockSpec((n, d), lambda i: (0, 0)),
            pl.BlockSpec((n, 1), lambda i: (0, 0)),
            pl.BlockSpec((1, 2 * d), lambda i: (0, 0)),
            pl.BlockSpec((1, 1), lambda i: (0, 0)),
        ],
        out_specs=pl.BlockSpec((NUM_GRAPHS, 2 * d), lambda i: (0, 0)),
        out_shape=jax.ShapeDtypeStruct((NUM_GRAPHS, 2 * d), jnp.float32),
    )(h, x, batch2d, wg2d, bg2d)


def kernel(x, edge_index, batch, Wl, W_ih, W_hh, b_ih, b_hh, Wg, bg):
    n, d = x.shape
    num_layers = Wl.shape[0]

    src2d = edge_index[0].reshape(-1, 128)
    dst2d = edge_index[1].reshape(-1, 128)
    wiht = W_ih.T
    whht = W_hh.T
    bih2 = b_ih.reshape(1, -1)
    bhh2 = b_hh.reshape(1, -1)
    batch2d = batch.reshape(n, 1)
    wg2d = Wg.reshape(1, -1)
    bg2d = bg.reshape(1, 1)

    h = x
    m = _matmul(x, Wl[0])
    for i in range(num_layers):
        parts = _edge_agg(m, src2d, dst2d)
        if i + 1 < num_layers:
            h, m = _gru(parts, h, wiht, whht, bih2, bhh2, Wl[i + 1])
        else:
            h = _gru(parts, h, wiht, whht, bih2, bhh2)
    return _pool(h, x, batch2d, wg2d, bg2d)


# speedup vs baseline: 1.3463x; 1.1260x over previous
import functools

import jax
import jax.numpy as jnp
from jax import lax
from jax.experimental import pallas as pl
from jax.experimental.pallas import tpu as pltpu
from jax.experimental.pallas import tpu_sc as plsc

NUM_GRAPHS = 64
_D = 128

_NC = 2
_NS = 16
_NW = _NC * _NS


def _make_edge_kernel(n_nodes, d, rows_pad, rp):
    rp0, rp1 = rp
    assert 16 * (rp0 + rp1) == rows_pad and rp0 % 8 == 0 and rp1 % 8 == 0
    units = n_nodes // 8
    assert n_nodes % 8 == 0
    upt = units // _NS
    urem = units % _NS
    rpt = upt * 8
    zr = max(z for z in range(8, 129, 8) if rpt % z == 0)
    nz = rpt // zr
    acc_rows = n_nodes + 16
    ch = 24

    mesh = plsc.VectorSubcoreMesh(core_axis_name="c", subcore_axis_name="s")

    @functools.partial(
        pl.kernel,
        mesh=mesh,
        out_type=jax.ShapeDtypeStruct((_NC, n_nodes, d), jnp.float32),
        scratch_types=[
            pltpu.VMEM((ch, 128), jnp.int32),
            pltpu.VMEM((ch, 128), jnp.int32),
            pltpu.VMEM((128, d), jnp.float32),
            pltpu.VMEM((128, d), jnp.float32),
            pltpu.VMEM_SHARED((acc_rows, d), jnp.float32),
            pltpu.SemaphoreType.DMA,
            pltpu.SemaphoreType.DMA,
        ],
    )
    def edge_kernel(m_hbm, src_hbm, dst_hbm, out_hbm,
                    srcv, dstv, rows0, rows1, acc, sem0, sem1):
        c = lax.axis_index("c")
        s = lax.axis_index("s")

        def zbody(i, _):
            r = i // (d // 16)
            k = (i % (d // 16)) * 16
            rows0[r, pl.ds(k, 16)] = jnp.zeros((16,), jnp.float32)
            return 0

        with jax.named_scope("ph1_zero"):
            lax.fori_loop(0, zr * (d // 16), zbody, 0)
            for j in range(nz):
                pltpu.sync_copy(rows0.at[pl.ds(0, zr)],
                                acc.at[pl.ds(s * rpt + j * zr, zr)])
            if urem:
                @pl.when(s < urem)
                def _zx():
                    pltpu.sync_copy(rows0.at[pl.ds(0, 8)],
                                    acc.at[pl.ds(_NS * rpt + s * 8, 8)])
            plsc.subcore_barrier()

        nrows = jnp.where(c == 0, rp0, rp1)
        row0 = jnp.where(c == 0, s * rp0, 16 * rp0 + s * rp1)

        def pair_body(cnt):
            def pair(p, _):
                j = 2 * p
                pltpu.async_copy(m_hbm.at[srcv.at[j + 1]], rows1, sem1)
                pltpu.make_async_copy(
                    m_hbm.at[srcv.at[j]], rows0, sem0).wait()
                pltpu.sync_copy(rows0, acc.at[dstv.at[j]], add=True)

                @pl.when(j + 2 < cnt)
                def _nxt():
                    pltpu.async_copy(m_hbm.at[srcv.at[j + 2]], rows0, sem0)

                pltpu.make_async_copy(
                    m_hbm.at[srcv.at[j + 1]], rows1, sem1).wait()
                pltpu.sync_copy(rows1, acc.at[dstv.at[j + 1]], add=True)
                return 0
            return pair

        def chunk_body(k, _):
            base = k * ch
            cnt = jnp.minimum(ch, nrows - base)
            pltpu.sync_copy(src_hbm.at[pl.ds(row0 + base, ch)], srcv)
            pltpu.sync_copy(dst_hbm.at[pl.ds(row0 + base, ch)], dstv)
            pltpu.async_copy(m_hbm.at[srcv.at[0]], rows0, sem0)
            lax.fori_loop(0, cnt // 2, pair_body(cnt), 0)
            return 0

        with jax.named_scope("ph2_edges"):
            lax.fori_loop(0, (nrows + ch - 1) // ch, chunk_body, 0)
            plsc.subcore_barrier()

        with jax.named_scope("ph3_dump"):
            pltpu.sync_copy(acc.at[pl.ds(s * rpt, rpt)],
                            out_hbm.at[c].at[pl.ds(s * rpt, rpt)])
            if urem:
                @pl.when(s < urem)
                def _dx():
                    o = _NS * rpt + s * 8
                    pltpu.sync_copy(acc.at[pl.ds(o, 8)],
                                    out_hbm.at[c].at[pl.ds(o, 8)])

    return edge_kernel


_CORE_SPLIT = (144, 16)


def _edge_agg(m, src2d, dst2d):
    n, d = m.shape
    rows_total = src2d.shape[0]
    unit = _NW * 8
    rows_pad = ((rows_total + unit - 1) // unit) * unit
    pad = rows_pad - rows_total + 24
    src2d = jnp.concatenate(
        [src2d, jnp.zeros((pad, 128), jnp.int32)], axis=0)
    dst2d = jnp.concatenate(
        [dst2d, jnp.full((pad, 128), n, jnp.int32)], axis=0)
    return _make_edge_kernel(n, d, rows_pad, _CORE_SPLIT)(m, src2d, dst2d)


def _mm_body(x_ref, w_ref, o_ref):
    o_ref[:] = jnp.dot(x_ref[:], w_ref[:], preferred_element_type=jnp.float32)


def _matmul(x, w):
    n, d = x.shape
    r = 1000
    return pl.pallas_call(
        _mm_body,
        grid=(n // r,),
        in_specs=[pl.BlockSpec((r, d), lambda i: (i, 0)),
                  pl.BlockSpec((d, d), lambda i: (0, 0))],
        out_specs=pl.BlockSpec((r, d), lambda i: (i, 0)),
        out_shape=jax.ShapeDtypeStruct((n, d), jnp.float32),
    )(x, w)


def _gru_body(p0, p1, h, wih, whh, bih, bhh, hout):
    d = _D
    agg = p0[:] + p1[:]
    gi = jnp.dot(agg, wih[:], preferred_element_type=jnp.float32) + bih[:]
    gh = jnp.dot(h[:], whh[:], preferred_element_type=jnp.float32) + bhh[:]
    r = jax.nn.sigmoid(gi[:, :d] + gh[:, :d])
    z = jax.nn.sigmoid(gi[:, d:2 * d] + gh[:, d:2 * d])
    n = jnp.tanh(gi[:, 2 * d:] + r * gh[:, 2 * d:])
    hout[:] = (1.0 - z) * n + z * h[:]


def _gru_fused_body(p0, p1, h, wih, whh, bih, bhh, wl, hout, mout):
    d = _D
    agg = p0[:] + p1[:]
    gi = jnp.dot(agg, wih[:], preferred_element_type=jnp.float32) + bih[:]
    gh = jnp.dot(h[:], whh[:], preferred_element_type=jnp.float32) + bhh[:]
    r = jax.nn.sigmoid(gi[:, :d] + gh[:, :d])
    z = jax.nn.sigmoid(gi[:, d:2 * d] + gh[:, d:2 * d])
    n = jnp.tanh(gi[:, 2 * d:] + r * gh[:, 2 * d:])
    hn = (1.0 - z) * n + z * h[:]
    hout[:] = hn
    mout[:] = jnp.dot(hn, wl[:], preferred_element_type=jnp.float32)


def _gru(parts, h, wiht, whht, bih2, bhh2, wl_next=None):
    n, d = h.shape
    r = 1000
    p0, p1 = parts[0], parts[1]
    row = lambda i: (i, 0)
    full = lambda i: (0, 0)
    in_specs = [
        pl.BlockSpec((r, d), row),
        pl.BlockSpec((r, d), row),
        pl.BlockSpec((r, d), row),
        pl.BlockSpec((d, 3 * d), full),
        pl.BlockSpec((d, 3 * d), full),
        pl.BlockSpec((1, 3 * d), full),
        pl.BlockSpec((1, 3 * d), full),
    ]
    if wl_next is None:
        return pl.pallas_call(
            _gru_body,
            grid=(n // r,),
            in_specs=in_specs,
            out_specs=pl.BlockSpec((r, d), row),
            out_shape=jax.ShapeDtypeStruct((n, d), jnp.float32),
        )(p0, p1, h, wiht, whht, bih2, bhh2)
    return pl.pallas_call(
        _gru_fused_body,
        grid=(n // r,),
        in_specs=in_specs + [pl.BlockSpec((d, d), full)],
        out_specs=[pl.BlockSpec((r, d), row), pl.BlockSpec((r, d), row)],
        out_shape=[jax.ShapeDtypeStruct((n, d), jnp.float32),
                   jax.ShapeDtypeStruct((n, d), jnp.float32)],
    )(p0, p1, h, wiht, whht, bih2, bhh2, wl_next)


def _pool_body(h_ref, x_ref, b_ref, wg_ref, bg_ref, o_ref):
    d = _D
    g = NUM_GRAPHS
    n = h_ref.shape[0]
    h = h_ref[:]
    x = x_ref[:]
    wg = wg_ref[:]
    gate = (jnp.sum(h * wg[:, :d], axis=1, keepdims=True)
            + jnp.sum(x * wg[:, d:], axis=1, keepdims=True)
            + bg_ref[0, 0])
    seg = lax.broadcasted_iota(jnp.int32, (n, g), 1)
    mask = b_ref[:] == seg
    maskf = mask.astype(jnp.float32)
    gm = jnp.max(jnp.where(mask, gate, -1e30), axis=0, keepdims=True)
    gmax_n = jnp.sum(maskf * gm, axis=1, keepdims=True)
    e = jnp.exp(gate - gmax_n)
    den = jnp.sum(maskf * e, axis=0, keepdims=True)
    den_n = jnp.sum(maskf * den, axis=1, keepdims=True)
    a = e / (den_n + 1e-16)
    attn = maskf * a
    dn = (((0,), (0,)), ((), ()))
    o_ref[:, :d] = lax.dot_general(attn, h, dn,
                                   preferred_element_type=jnp.float32)
    o_ref[:, d:] = lax.dot_general(attn, x, dn,
                                   preferred_element_type=jnp.float32)


def _pool(h, x, batch2d, wg2d, bg2d):
    n, d = h.shape
    full = lambda: (0, 0)
    return pl.pallas_call(
        _pool_body,
        grid=(1,),
        in_specs=[
            pl.BlockSpec((n, d), lambda i: (0, 0)),
            pl.BlockSpec((n, d), lambda i: (0, 0)),
            pl.BlockSpec((n, 1), lambda i: (0, 0)),
            pl.BlockSpec((1, 2 * d), lambda i: (0, 0)),
            pl.BlockSpec((1, 1), lambda i: (0, 0)),
        ],
        out_specs=pl.BlockSpec((NUM_GRAPHS, 2 * d), lambda i: (0, 0)),
        out_shape=jax.ShapeDtypeStruct((NUM_GRAPHS, 2 * d), jnp.float32),
    )(h, x, batch2d, wg2d, bg2d)


def kernel(x, edge_index, batch, Wl, W_ih, W_hh, b_ih, b_hh, Wg, bg):
    n, d = x.shape
    num_layers = Wl.shape[0]

    src2d = edge_index[0].reshape(-1, 128)
    dst2d = edge_index[1].reshape(-1, 128)
    wiht = W_ih.T
    whht = W_hh.T
    bih2 = b_ih.reshape(1, -1)
    bhh2 = b_hh.reshape(1, -1)
    batch2d = batch.reshape(n, 1)
    wg2d = Wg.reshape(1, -1)
    bg2d = bg.reshape(1, 1)

    h = x
    m = _matmul(x, Wl[0])
    for i in range(num_layers):
        parts = _edge_agg(m, src2d, dst2d)
        if i + 1 < num_layers:
            h, m = _gru(parts, h, wiht, whht, bih2, bhh2, Wl[i + 1])
        else:
            h = _gru(parts, h, wiht, whht, bih2, bhh2)
    return _pool(h, x, batch2d, wg2d, bg2d)


# speedup vs baseline: 1.3642x; 1.0132x over previous
import functools

import jax
import jax.numpy as jnp
from jax import lax
from jax.experimental import pallas as pl
from jax.experimental.pallas import tpu as pltpu
from jax.experimental.pallas import tpu_sc as plsc

NUM_GRAPHS = 64
_D = 128

_NC = 2
_NS = 16
_NW = _NC * _NS


def _make_edge_kernel(n_nodes, d, rows_pad, rp):
    rp0, rp1 = rp
    assert 16 * (rp0 + rp1) == rows_pad and rp0 % 8 == 0 and rp1 % 8 == 0
    units = n_nodes // 8
    assert n_nodes % 8 == 0
    upt = units // _NS
    urem = units % _NS
    rpt = upt * 8
    zr = max(z for z in range(8, 129, 8) if rpt % z == 0)
    nz = rpt // zr
    acc_rows = n_nodes + 16
    ch = 24

    mesh = plsc.VectorSubcoreMesh(core_axis_name="c", subcore_axis_name="s")

    @functools.partial(
        pl.kernel,
        mesh=mesh,
        out_type=jax.ShapeDtypeStruct((_NC, n_nodes, d), jnp.float32),
        scratch_types=[
            pltpu.VMEM((ch, 128), jnp.int32),
            pltpu.VMEM((ch, 128), jnp.int32),
            pltpu.VMEM((128, d), jnp.float32),
            pltpu.VMEM((128, d), jnp.float32),
            pltpu.VMEM_SHARED((acc_rows, d), jnp.float32),
            pltpu.SemaphoreType.DMA,
            pltpu.SemaphoreType.DMA,
        ],
    )
    def edge_kernel(m_hbm, src_hbm, dst_hbm, out_hbm,
                    srcv, dstv, rows0, rows1, acc, sem0, sem1):
        c = lax.axis_index("c")
        s = lax.axis_index("s")

        def zbody(i, _):
            r = i // (d // 16)
            k = (i % (d // 16)) * 16
            rows0[r, pl.ds(k, 16)] = jnp.zeros((16,), jnp.float32)
            return 0

        with jax.named_scope("ph1_zero"):
            lax.fori_loop(0, zr * (d // 16), zbody, 0)
            for j in range(nz):
                pltpu.sync_copy(rows0.at[pl.ds(0, zr)],
                                acc.at[pl.ds(s * rpt + j * zr, zr)])
            if urem:
                @pl.when(s < urem)
                def _zx():
                    pltpu.sync_copy(rows0.at[pl.ds(0, 8)],
                                    acc.at[pl.ds(_NS * rpt + s * 8, 8)])
            plsc.subcore_barrier()

        nrows = jnp.where(c == 0, rp0, rp1)
        row0 = jnp.where(c == 0, s * rp0, 16 * rp0 + s * rp1)

        def pair_body(cnt):
            def pair(p, _):
                j = 2 * p
                pltpu.async_copy(m_hbm.at[srcv.at[j + 1]], rows1, sem1)
                pltpu.make_async_copy(
                    m_hbm.at[srcv.at[j]], rows0, sem0).wait()
                pltpu.sync_copy(rows0, acc.at[dstv.at[j]], add=True)

                @pl.when(j + 2 < cnt)
                def _nxt():
                    pltpu.async_copy(m_hbm.at[srcv.at[j + 2]], rows0, sem0)

                pltpu.make_async_copy(
                    m_hbm.at[srcv.at[j + 1]], rows1, sem1).wait()
                pltpu.sync_copy(rows1, acc.at[dstv.at[j + 1]], add=True)
                return 0
            return pair

        def chunk_body(k, _):
            base = k * ch
            cnt = jnp.minimum(ch, nrows - base)
            pltpu.sync_copy(src_hbm.at[pl.ds(row0 + base, ch)], srcv)
            pltpu.sync_copy(dst_hbm.at[pl.ds(row0 + base, ch)], dstv)
            pltpu.async_copy(m_hbm.at[srcv.at[0]], rows0, sem0)
            lax.fori_loop(0, cnt // 2, pair_body(cnt), 0)
            return 0

        with jax.named_scope("ph2_edges"):
            lax.fori_loop(0, (nrows + ch - 1) // ch, chunk_body, 0)
            plsc.subcore_barrier()

        with jax.named_scope("ph3_dump"):
            pltpu.sync_copy(acc.at[pl.ds(s * rpt, rpt)],
                            out_hbm.at[c].at[pl.ds(s * rpt, rpt)])
            if urem:
                @pl.when(s < urem)
                def _dx():
                    o = _NS * rpt + s * 8
                    pltpu.sync_copy(acc.at[pl.ds(o, 8)],
                                    out_hbm.at[c].at[pl.ds(o, 8)])

    return edge_kernel


_CORE_SPLIT = (152, 8)


def _edge_agg(m, src2d, dst2d):
    n, d = m.shape
    rows_total = src2d.shape[0]
    unit = _NW * 8
    rows_pad = ((rows_total + unit - 1) // unit) * unit
    pad = rows_pad - rows_total + 24
    src2d = jnp.concatenate(
        [src2d, jnp.zeros((pad, 128), jnp.int32)], axis=0)
    dst2d = jnp.concatenate(
        [dst2d, jnp.full((pad, 128), n, jnp.int32)], axis=0)
    return _make_edge_kernel(n, d, rows_pad, _CORE_SPLIT)(m, src2d, dst2d)


def _mm_body(x_ref, w_ref, o_ref):
    o_ref[:] = jnp.dot(x_ref[:], w_ref[:], preferred_element_type=jnp.float32)


def _matmul(x, w):
    n, d = x.shape
    r = 1000
    return pl.pallas_call(
        _mm_body,
        grid=(n // r,),
        in_specs=[pl.BlockSpec((r, d), lambda i: (i, 0)),
                  pl.BlockSpec((d, d), lambda i: (0, 0))],
        out_specs=pl.BlockSpec((r, d), lambda i: (i, 0)),
        out_shape=jax.ShapeDtypeStruct((n, d), jnp.float32),
    )(x, w)


def _gru_body(p0, p1, h, wih, whh, bih, bhh, hout):
    d = _D
    agg = p0[:] + p1[:]
    gi = jnp.dot(agg, wih[:], preferred_element_type=jnp.float32) + bih[:]
    gh = jnp.dot(h[:], whh[:], preferred_element_type=jnp.float32) + bhh[:]
    r = jax.nn.sigmoid(gi[:, :d] + gh[:, :d])
    z = jax.nn.sigmoid(gi[:, d:2 * d] + gh[:, d:2 * d])
    n = jnp.tanh(gi[:, 2 * d:] + r * gh[:, 2 * d:])
    hout[:] = (1.0 - z) * n + z * h[:]


def _gru_fused_body(p0, p1, h, wih, whh, bih, bhh, wl, hout, mout):
    d = _D
    agg = p0[:] + p1[:]
    gi = jnp.dot(agg, wih[:], preferred_element_type=jnp.float32) + bih[:]
    gh = jnp.dot(h[:], whh[:], preferred_element_type=jnp.float32) + bhh[:]
    r = jax.nn.sigmoid(gi[:, :d] + gh[:, :d])
    z = jax.nn.sigmoid(gi[:, d:2 * d] + gh[:, d:2 * d])
    n = jnp.tanh(gi[:, 2 * d:] + r * gh[:, 2 * d:])
    hn = (1.0 - z) * n + z * h[:]
    hout[:] = hn
    mout[:] = jnp.dot(hn, wl[:], preferred_element_type=jnp.float32)


def _gru(parts, h, wiht, whht, bih2, bhh2, wl_next=None):
    n, d = h.shape
    r = 1000
    p0, p1 = parts[0], parts[1]
    row = lambda i: (i, 0)
    full = lambda i: (0, 0)
    in_specs = [
        pl.BlockSpec((r, d), row),
        pl.BlockSpec((r, d), row),
        pl.BlockSpec((r, d), row),
        pl.BlockSpec((d, 3 * d), full),
        pl.BlockSpec((d, 3 * d), full),
        pl.BlockSpec((1, 3 * d), full),
        pl.BlockSpec((1, 3 * d), full),
    ]
    if wl_next is None:
        return pl.pallas_call(
            _gru_body,
            grid=(n // r,),
            in_specs=in_specs,
            out_specs=pl.BlockSpec((r, d), row),
            out_shape=jax.ShapeDtypeStruct((n, d), jnp.float32),
        )(p0, p1, h, wiht, whht, bih2, bhh2)
    return pl.pallas_call(
        _gru_fused_body,
        grid=(n // r,),
        in_specs=in_specs + [pl.BlockSpec((d, d), full)],
        out_specs=[pl.BlockSpec((r, d), row), pl.BlockSpec((r, d), row)],
        out_shape=[jax.ShapeDtypeStruct((n, d), jnp.float32),
                   jax.ShapeDtypeStruct((n, d), jnp.float32)],
    )(p0, p1, h, wiht, whht, bih2, bhh2, wl_next)


def _pool_body(h_ref, x_ref, b_ref, wg_ref, bg_ref, o_ref):
    d = _D
    g = NUM_GRAPHS
    n = h_ref.shape[0]
    h = h_ref[:]
    x = x_ref[:]
    wg = wg_ref[:]
    gate = (jnp.sum(h * wg[:, :d], axis=1, keepdims=True)
            + jnp.sum(x * wg[:, d:], axis=1, keepdims=True)
            + bg_ref[0, 0])
    seg = lax.broadcasted_iota(jnp.int32, (n, g), 1)
    mask = b_ref[:] == seg
    maskf = mask.astype(jnp.float32)
    gm = jnp.max(jnp.where(mask, gate, -1e30), axis=0, keepdims=True)
    gmax_n = jnp.sum(maskf * gm, axis=1, keepdims=True)
    e = jnp.exp(gate - gmax_n)
    den = jnp.sum(maskf * e, axis=0, keepdims=True)
    den_n = jnp.sum(maskf * den, axis=1, keepdims=True)
    a = e / (den_n + 1e-16)
    attn = maskf * a
    dn = (((0,), (0,)), ((), ()))
    o_ref[:, :d] = lax.dot_general(attn, h, dn,
                                   preferred_element_type=jnp.float32)
    o_ref[:, d:] = lax.dot_general(attn, x, dn,
                                   preferred_element_type=jnp.float32)


def _pool(h, x, batch2d, wg2d, bg2d):
    n, d = h.shape
    full = lambda: (0, 0)
    return pl.pallas_call(
        _pool_body,
        grid=(1,),
        in_specs=[
            pl.BlockSpec((n, d), lambda i: (0, 0)),
            pl.BlockSpec((n, d), lambda i: (0, 0)),
            pl.BlockSpec((n, 1), lambda i: (0, 0)),
            pl.BlockSpec((1, 2 * d), lambda i: (0, 0)),
            pl.BlockSpec((1, 1), lambda i: (0, 0)),
        ],
        out_specs=pl.BlockSpec((NUM_GRAPHS, 2 * d), lambda i: (0, 0)),
        out_shape=jax.ShapeDtypeStruct((NUM_GRAPHS, 2 * d), jnp.float32),
    )(h, x, batch2d, wg2d, bg2d)


def kernel(x, edge_index, batch, Wl, W_ih, W_hh, b_ih, b_hh, Wg, bg):
    n, d = x.shape
    num_layers = Wl.shape[0]

    src2d = edge_index[0].reshape(-1, 128)
    dst2d = edge_index[1].reshape(-1, 128)
    wiht = W_ih.T
    whht = W_hh.T
    bih2 = b_ih.reshape(1, -1)
    bhh2 = b_hh.reshape(1, -1)
    batch2d = batch.reshape(n, 1)
    wg2d = Wg.reshape(1, -1)
    bg2d = bg.reshape(1, 1)

    h = x
    m = _matmul(x, Wl[0])
    for i in range(num_layers):
        parts = _edge_agg(m, src2d, dst2d)
        if i + 1 < num_layers:
            h, m = _gru(parts, h, wiht, whht, bih2, bhh2, Wl[i + 1])
        else:
            h = _gru(parts, h, wiht, whht, bih2, bhh2)
    return _pool(h, x, batch2d, wg2d, bg2d)


# speedup vs baseline: 2.2876x; 1.6769x over previous
import functools

import jax
import jax.numpy as jnp
from jax import lax
from jax.experimental import pallas as pl
from jax.experimental.pallas import tpu as pltpu
from jax.experimental.pallas import tpu_sc as plsc

NUM_GRAPHS = 64
_D = 128

_NC = 2
_NS = 16
_NW = _NC * _NS


def _make_edge_kernel(n_nodes, d, rows_pad, rp):
    rp0, rp1 = rp
    assert 16 * (rp0 + rp1) == rows_pad and rp0 % 8 == 0 and rp1 % 8 == 0
    units = n_nodes // 8
    assert n_nodes % 8 == 0
    upt = units // _NS
    urem = units % _NS
    rpt = upt * 8
    zr = max(z for z in range(8, 129, 8) if rpt % z == 0)
    nz = rpt // zr
    acc_rows = n_nodes + 16
    ch = 24

    mesh = plsc.VectorSubcoreMesh(core_axis_name="c", subcore_axis_name="s")

    @functools.partial(
        pl.kernel,
        mesh=mesh,
        out_type=jax.ShapeDtypeStruct((_NC, n_nodes, d), jnp.float32),
        scratch_types=[
            pltpu.VMEM((ch, 128), jnp.int32),
            pltpu.VMEM((ch, 128), jnp.int32),
            pltpu.VMEM((128, d), jnp.float32),
            pltpu.VMEM((128, d), jnp.float32),
            pltpu.VMEM_SHARED((acc_rows, d), jnp.float32),
            pltpu.SemaphoreType.DMA,
            pltpu.SemaphoreType.DMA,
        ],
    )
    def edge_kernel(m_hbm, src_hbm, dst_hbm, out_hbm,
                    srcv, dstv, rows0, rows1, acc, sem0, sem1):
        c = lax.axis_index("c")
        s = lax.axis_index("s")

        def zbody(i, _):
            r = i // (d // 16)
            k = (i % (d // 16)) * 16
            rows0[r, pl.ds(k, 16)] = jnp.zeros((16,), jnp.float32)
            return 0

        with jax.named_scope("ph1_zero"):
            lax.fori_loop(0, zr * (d // 16), zbody, 0)
            for j in range(nz):
                pltpu.sync_copy(rows0.at[pl.ds(0, zr)],
                                acc.at[pl.ds(s * rpt + j * zr, zr)])
            if urem:
                @pl.when(s < urem)
                def _zx():
                    pltpu.sync_copy(rows0.at[pl.ds(0, 8)],
                                    acc.at[pl.ds(_NS * rpt + s * 8, 8)])
            plsc.subcore_barrier()

        nrows = jnp.where(c == 0, rp0, rp1)
        row0 = jnp.where(c == 0, s * rp0, 16 * rp0 + s * rp1)

        def pair_body(cnt):
            def pair(p, _):
                j = 2 * p
                pltpu.async_copy(m_hbm.at[srcv.at[j + 1]], rows1, sem1)
                pltpu.make_async_copy(
                    m_hbm.at[srcv.at[j]], rows0, sem0).wait()
                pltpu.sync_copy(rows0, acc.at[dstv.at[j]], add=True)

                @pl.when(j + 2 < cnt)
                def _nxt():
                    pltpu.async_copy(m_hbm.at[srcv.at[j + 2]], rows0, sem0)

                pltpu.make_async_copy(
                    m_hbm.at[srcv.at[j + 1]], rows1, sem1).wait()
                pltpu.sync_copy(rows1, acc.at[dstv.at[j + 1]], add=True)
                return 0
            return pair

        def chunk_body(k, _):
            base = k * ch
            cnt = jnp.minimum(ch, nrows - base)
            pltpu.sync_copy(src_hbm.at[pl.ds(row0 + base, ch)], srcv)
            pltpu.sync_copy(dst_hbm.at[pl.ds(row0 + base, ch)], dstv)
            pltpu.async_copy(m_hbm.at[srcv.at[0]], rows0, sem0)
            lax.fori_loop(0, cnt // 2, pair_body(cnt), 0)
            return 0

        with jax.named_scope("ph2_edges"):
            lax.fori_loop(0, (nrows + ch - 1) // ch, chunk_body, 0)
            plsc.subcore_barrier()

        with jax.named_scope("ph3_dump"):
            pltpu.sync_copy(acc.at[pl.ds(s * rpt, rpt)],
                            out_hbm.at[c].at[pl.ds(s * rpt, rpt)])
            if urem:
                @pl.when(s < urem)
                def _dx():
                    o = _NS * rpt + s * 8
                    pltpu.sync_copy(acc.at[pl.ds(o, 8)],
                                    out_hbm.at[c].at[pl.ds(o, 8)])

    return edge_kernel


_CORE_SPLIT = (152, 8)


def _edge_agg(m, src2d, dst2d):
    n, d = m.shape
    rows_total = src2d.shape[0]
    unit = _NW * 8
    rows_pad = ((rows_total + unit - 1) // unit) * unit
    pad = rows_pad - rows_total + 24
    lane = jnp.arange(128, dtype=jnp.int32)
    src_pad = jnp.broadcast_to((lane * 79) % n, (pad, 128))
    dst_pad = jnp.broadcast_to(n + (lane % 16), (pad, 128))
    src2d = jnp.concatenate([src2d, src_pad], axis=0)
    dst2d = jnp.concatenate([dst2d, dst_pad], axis=0)
    return _make_edge_kernel(n, d, rows_pad, _CORE_SPLIT)(m, src2d, dst2d)


def _mm_body(x_ref, w_ref, o_ref):
    o_ref[:] = jnp.dot(x_ref[:], w_ref[:], preferred_element_type=jnp.float32)


def _matmul(x, w):
    n, d = x.shape
    r = 1000
    return pl.pallas_call(
        _mm_body,
        grid=(n // r,),
        in_specs=[pl.BlockSpec((r, d), lambda i: (i, 0)),
                  pl.BlockSpec((d, d), lambda i: (0, 0))],
        out_specs=pl.BlockSpec((r, d), lambda i: (i, 0)),
        out_shape=jax.ShapeDtypeStruct((n, d), jnp.float32),
    )(x, w)


def _gru_body(p0, p1, h, wih, whh, bih, bhh, hout):
    d = _D
    agg = p0[:] + p1[:]
    gi = jnp.dot(agg, wih[:], preferred_element_type=jnp.float32) + bih[:]
    gh = jnp.dot(h[:], whh[:], preferred_element_type=jnp.float32) + bhh[:]
    r = jax.nn.sigmoid(gi[:, :d] + gh[:, :d])
    z = jax.nn.sigmoid(gi[:, d:2 * d] + gh[:, d:2 * d])
    n = jnp.tanh(gi[:, 2 * d:] + r * gh[:, 2 * d:])
    hout[:] = (1.0 - z) * n + z * h[:]


def _gru_fused_body(p0, p1, h, wih, whh, bih, bhh, wl, hout, mout):
    d = _D
    agg = p0[:] + p1[:]
    gi = jnp.dot(agg, wih[:], preferred_element_type=jnp.float32) + bih[:]
    gh = jnp.dot(h[:], whh[:], preferred_element_type=jnp.float32) + bhh[:]
    r = jax.nn.sigmoid(gi[:, :d] + gh[:, :d])
    z = jax.nn.sigmoid(gi[:, d:2 * d] + gh[:, d:2 * d])
    n = jnp.tanh(gi[:, 2 * d:] + r * gh[:, 2 * d:])
    hn = (1.0 - z) * n + z * h[:]
    hout[:] = hn
    mout[:] = jnp.dot(hn, wl[:], preferred_element_type=jnp.float32)


def _gru(parts, h, wiht, whht, bih2, bhh2, wl_next=None):
    n, d = h.shape
    r = 1000
    p0, p1 = parts[0], parts[1]
    row = lambda i: (i, 0)
    full = lambda i: (0, 0)
    in_specs = [
        pl.BlockSpec((r, d), row),
        pl.BlockSpec((r, d), row),
        pl.BlockSpec((r, d), row),
        pl.BlockSpec((d, 3 * d), full),
        pl.BlockSpec((d, 3 * d), full),
        pl.BlockSpec((1, 3 * d), full),
        pl.BlockSpec((1, 3 * d), full),
    ]
    if wl_next is None:
        return pl.pallas_call(
            _gru_body,
            grid=(n // r,),
            in_specs=in_specs,
            out_specs=pl.BlockSpec((r, d), row),
            out_shape=jax.ShapeDtypeStruct((n, d), jnp.float32),
        )(p0, p1, h, wiht, whht, bih2, bhh2)
    return pl.pallas_call(
        _gru_fused_body,
        grid=(n // r,),
        in_specs=in_specs + [pl.BlockSpec((d, d), full)],
        out_specs=[pl.BlockSpec((r, d), row), pl.BlockSpec((r, d), row)],
        out_shape=[jax.ShapeDtypeStruct((n, d), jnp.float32),
                   jax.ShapeDtypeStruct((n, d), jnp.float32)],
    )(p0, p1, h, wiht, whht, bih2, bhh2, wl_next)


def _pool_body(h_ref, x_ref, b_ref, wg_ref, bg_ref, o_ref):
    d = _D
    g = NUM_GRAPHS
    n = h_ref.shape[0]
    h = h_ref[:]
    x = x_ref[:]
    wg = wg_ref[:]
    gate = (jnp.sum(h * wg[:, :d], axis=1, keepdims=True)
            + jnp.sum(x * wg[:, d:], axis=1, keepdims=True)
            + bg_ref[0, 0])
    seg = lax.broadcasted_iota(jnp.int32, (n, g), 1)
    mask = b_ref[:] == seg
    maskf = mask.astype(jnp.float32)
    gm = jnp.max(jnp.where(mask, gate, -1e30), axis=0, keepdims=True)
    gmax_n = jnp.sum(maskf * gm, axis=1, keepdims=True)
    e = jnp.exp(gate - gmax_n)
    den = jnp.sum(maskf * e, axis=0, keepdims=True)
    den_n = jnp.sum(maskf * den, axis=1, keepdims=True)
    a = e / (den_n + 1e-16)
    attn = maskf * a
    dn = (((0,), (0,)), ((), ()))
    o_ref[:, :d] = lax.dot_general(attn, h, dn,
                                   preferred_element_type=jnp.float32)
    o_ref[:, d:] = lax.dot_general(attn, x, dn,
                                   preferred_element_type=jnp.float32)


def _pool(h, x, batch2d, wg2d, bg2d):
    n, d = h.shape
    full = lambda: (0, 0)
    return pl.pallas_call(
        _pool_body,
        grid=(1,),
        in_specs=[
            pl.BlockSpec((n, d), lambda i: (0, 0)),
            pl.BlockSpec((n, d), lambda i: (0, 0)),
            pl.BlockSpec((n, 1), lambda i: (0, 0)),
            pl.BlockSpec((1, 2 * d), lambda i: (0, 0)),
            pl.BlockSpec((1, 1), lambda i: (0, 0)),
        ],
        out_specs=pl.BlockSpec((NUM_GRAPHS, 2 * d), lambda i: (0, 0)),
        out_shape=jax.ShapeDtypeStruct((NUM_GRAPHS, 2 * d), jnp.float32),
    )(h, x, batch2d, wg2d, bg2d)


def kernel(x, edge_index, batch, Wl, W_ih, W_hh, b_ih, b_hh, Wg, bg):
    n, d = x.shape
    num_layers = Wl.shape[0]

    src2d = edge_index[0].reshape(-1, 128)
    dst2d = edge_index[1].reshape(-1, 128)
    wiht = W_ih.T
    whht = W_hh.T
    bih2 = b_ih.reshape(1, -1)
    bhh2 = b_hh.reshape(1, -1)
    batch2d = batch.reshape(n, 1)
    wg2d = Wg.reshape(1, -1)
    bg2d = bg.reshape(1, 1)

    h = x
    m = _matmul(x, Wl[0])
    for i in range(num_layers):
        parts = _edge_agg(m, src2d, dst2d)
        if i + 1 < num_layers:
            h, m = _gru(parts, h, wiht, whht, bih2, bhh2, Wl[i + 1])
        else:
            h = _gru(parts, h, wiht, whht, bih2, bhh2)
    return _pool(h, x, batch2d, wg2d, bg2d)


# speedup vs baseline: 3.4778x; 1.5203x over previous
import functools

import jax
import jax.numpy as jnp
from jax import lax
from jax.experimental import pallas as pl
from jax.experimental.pallas import tpu as pltpu
from jax.experimental.pallas import tpu_sc as plsc

NUM_GRAPHS = 64
_D = 128

_NC = 2
_NS = 16
_NW = _NC * _NS


def _make_edge_kernel(n_nodes, d, rows_pad, rp):
    rp0, rp1 = rp
    assert 16 * (rp0 + rp1) == rows_pad and rp0 % 8 == 0 and rp1 % 8 == 0
    units = n_nodes // 8
    assert n_nodes % 8 == 0
    upt = units // _NS
    urem = units % _NS
    rpt = upt * 8
    zr = max(z for z in range(8, 129, 8) if rpt % z == 0)
    nz = rpt // zr
    acc_rows = n_nodes + 16
    ch = 24

    mesh = plsc.VectorSubcoreMesh(core_axis_name="c", subcore_axis_name="s")

    @functools.partial(
        pl.kernel,
        mesh=mesh,
        out_type=jax.ShapeDtypeStruct((_NC, n_nodes, d), jnp.float32),
        scratch_types=[
            pltpu.VMEM((ch, 128), jnp.int32),
            pltpu.VMEM((ch, 128), jnp.int32),
            pltpu.VMEM((128, d), jnp.float32),
            pltpu.VMEM((128, d), jnp.float32),
            pltpu.VMEM_SHARED((acc_rows, d), jnp.float32),
            pltpu.SemaphoreType.DMA,
            pltpu.SemaphoreType.DMA,
        ],
    )
    def edge_kernel(m_hbm, src_hbm, dst_hbm, out_hbm,
                    srcv, dstv, rows0, rows1, acc, sem0, sem1):
        c = lax.axis_index("c")
        s = lax.axis_index("s")

        def zbody(i, _):
            r = i // (d // 16)
            k = (i % (d // 16)) * 16
            rows0[r, pl.ds(k, 16)] = jnp.zeros((16,), jnp.float32)
            return 0

        with jax.named_scope("ph1_zero"):
            lax.fori_loop(0, zr * (d // 16), zbody, 0)
            for j in range(nz):
                pltpu.sync_copy(rows0.at[pl.ds(0, zr)],
                                acc.at[pl.ds(s * rpt + j * zr, zr)])
            if urem:
                @pl.when(s < urem)
                def _zx():
                    pltpu.sync_copy(rows0.at[pl.ds(0, 8)],
                                    acc.at[pl.ds(_NS * rpt + s * 8, 8)])
            plsc.subcore_barrier()

        nrows = jnp.where(c == 0, rp0, rp1)
        row0 = jnp.where(c == 0, s * rp0, 16 * rp0 + s * rp1)

        def pair_body(cnt):
            def pair(p, _):
                j = 2 * p
                pltpu.async_copy(m_hbm.at[srcv.at[j + 1]], rows1, sem1)
                pltpu.make_async_copy(
                    m_hbm.at[srcv.at[j]], rows0, sem0).wait()
                pltpu.sync_copy(rows0, acc.at[dstv.at[j]], add=True)

                @pl.when(j + 2 < cnt)
                def _nxt():
                    pltpu.async_copy(m_hbm.at[srcv.at[j + 2]], rows0, sem0)

                pltpu.make_async_copy(
                    m_hbm.at[srcv.at[j + 1]], rows1, sem1).wait()
                pltpu.sync_copy(rows1, acc.at[dstv.at[j + 1]], add=True)
                return 0
            return pair

        def chunk_body(k, _):
            base = k * ch
            cnt = jnp.minimum(ch, nrows - base)
            pltpu.sync_copy(src_hbm.at[pl.ds(row0 + base, ch)], srcv)
            pltpu.sync_copy(dst_hbm.at[pl.ds(row0 + base, ch)], dstv)
            pltpu.async_copy(m_hbm.at[srcv.at[0]], rows0, sem0)
            lax.fori_loop(0, cnt // 2, pair_body(cnt), 0)
            return 0

        with jax.named_scope("ph2_edges"):
            lax.fori_loop(0, (nrows + ch - 1) // ch, chunk_body, 0)
            plsc.subcore_barrier()

        with jax.named_scope("ph3_dump"):
            pltpu.sync_copy(acc.at[pl.ds(s * rpt, rpt)],
                            out_hbm.at[c].at[pl.ds(s * rpt, rpt)])
            if urem:
                @pl.when(s < urem)
                def _dx():
                    o = _NS * rpt + s * 8
                    pltpu.sync_copy(acc.at[pl.ds(o, 8)],
                                    out_hbm.at[c].at[pl.ds(o, 8)])

    return edge_kernel


_CORE_SPLIT = (80, 80)


def _edge_agg(m, src2d, dst2d):
    n, d = m.shape
    rows_total = src2d.shape[0]
    unit = _NW * 8
    rows_pad = ((rows_total + unit - 1) // unit) * unit
    pad = rows_pad - rows_total + 24
    lane = jnp.arange(128, dtype=jnp.int32)
    src_pad = jnp.broadcast_to((lane * 79) % n, (pad, 128))
    dst_pad = jnp.broadcast_to(n + (lane % 16), (pad, 128))
    src2d = jnp.concatenate([src2d, src_pad], axis=0)
    dst2d = jnp.concatenate([dst2d, dst_pad], axis=0)
    return _make_edge_kernel(n, d, rows_pad, _CORE_SPLIT)(m, src2d, dst2d)


def _mm_body(x_ref, w_ref, o_ref):
    o_ref[:] = jnp.dot(x_ref[:], w_ref[:], preferred_element_type=jnp.float32)


def _matmul(x, w):
    n, d = x.shape
    r = 1000
    return pl.pallas_call(
        _mm_body,
        grid=(n // r,),
        in_specs=[pl.BlockSpec((r, d), lambda i: (i, 0)),
                  pl.BlockSpec((d, d), lambda i: (0, 0))],
        out_specs=pl.BlockSpec((r, d), lambda i: (i, 0)),
        out_shape=jax.ShapeDtypeStruct((n, d), jnp.float32),
    )(x, w)


def _gru_body(p0, p1, h, wih, whh, bih, bhh, hout):
    d = _D
    agg = p0[:] + p1[:]
    gi = jnp.dot(agg, wih[:], preferred_element_type=jnp.float32) + bih[:]
    gh = jnp.dot(h[:], whh[:], preferred_element_type=jnp.float32) + bhh[:]
    r = jax.nn.sigmoid(gi[:, :d] + gh[:, :d])
    z = jax.nn.sigmoid(gi[:, d:2 * d] + gh[:, d:2 * d])
    n = jnp.tanh(gi[:, 2 * d:] + r * gh[:, 2 * d:])
    hout[:] = (1.0 - z) * n + z * h[:]


def _gru_fused_body(p0, p1, h, wih, whh, bih, bhh, wl, hout, mout):
    d = _D
    agg = p0[:] + p1[:]
    gi = jnp.dot(agg, wih[:], preferred_element_type=jnp.float32) + bih[:]
    gh = jnp.dot(h[:], whh[:], preferred_element_type=jnp.float32) + bhh[:]
    r = jax.nn.sigmoid(gi[:, :d] + gh[:, :d])
    z = jax.nn.sigmoid(gi[:, d:2 * d] + gh[:, d:2 * d])
    n = jnp.tanh(gi[:, 2 * d:] + r * gh[:, 2 * d:])
    hn = (1.0 - z) * n + z * h[:]
    hout[:] = hn
    mout[:] = jnp.dot(hn, wl[:], preferred_element_type=jnp.float32)


def _gru(parts, h, wiht, whht, bih2, bhh2, wl_next=None):
    n, d = h.shape
    r = 1000
    p0, p1 = parts[0], parts[1]
    row = lambda i: (i, 0)
    full = lambda i: (0, 0)
    in_specs = [
        pl.BlockSpec((r, d), row),
        pl.BlockSpec((r, d), row),
        pl.BlockSpec((r, d), row),
        pl.BlockSpec((d, 3 * d), full),
        pl.BlockSpec((d, 3 * d), full),
        pl.BlockSpec((1, 3 * d), full),
        pl.BlockSpec((1, 3 * d), full),
    ]
    if wl_next is None:
        return pl.pallas_call(
            _gru_body,
            grid=(n // r,),
            in_specs=in_specs,
            out_specs=pl.BlockSpec((r, d), row),
            out_shape=jax.ShapeDtypeStruct((n, d), jnp.float32),
        )(p0, p1, h, wiht, whht, bih2, bhh2)
    return pl.pallas_call(
        _gru_fused_body,
        grid=(n // r,),
        in_specs=in_specs + [pl.BlockSpec((d, d), full)],
        out_specs=[pl.BlockSpec((r, d), row), pl.BlockSpec((r, d), row)],
        out_shape=[jax.ShapeDtypeStruct((n, d), jnp.float32),
                   jax.ShapeDtypeStruct((n, d), jnp.float32)],
    )(p0, p1, h, wiht, whht, bih2, bhh2, wl_next)


def _pool_body(h_ref, x_ref, b_ref, wg_ref, bg_ref, o_ref):
    d = _D
    g = NUM_GRAPHS
    n = h_ref.shape[0]
    h = h_ref[:]
    x = x_ref[:]
    wg = wg_ref[:]
    gate = (jnp.sum(h * wg[:, :d], axis=1, keepdims=True)
            + jnp.sum(x * wg[:, d:], axis=1, keepdims=True)
            + bg_ref[0, 0])
    seg = lax.broadcasted_iota(jnp.int32, (n, g), 1)
    mask = b_ref[:] == seg
    maskf = mask.astype(jnp.float32)
    gm = jnp.max(jnp.where(mask, gate, -1e30), axis=0, keepdims=True)
    gmax_n = jnp.sum(maskf * gm, axis=1, keepdims=True)
    e = jnp.exp(gate - gmax_n)
    den = jnp.sum(maskf * e, axis=0, keepdims=True)
    den_n = jnp.sum(maskf * den, axis=1, keepdims=True)
    a = e / (den_n + 1e-16)
    attn = maskf * a
    dn = (((0,), (0,)), ((), ()))
    o_ref[:, :d] = lax.dot_general(attn, h, dn,
                                   preferred_element_type=jnp.float32)
    o_ref[:, d:] = lax.dot_general(attn, x, dn,
                                   preferred_element_type=jnp.float32)


def _pool(h, x, batch2d, wg2d, bg2d):
    n, d = h.shape
    full = lambda: (0, 0)
    return pl.pallas_call(
        _pool_body,
        grid=(1,),
        in_specs=[
            pl.BlockSpec((n, d), lambda i: (0, 0)),
            pl.BlockSpec((n, d), lambda i: (0, 0)),
            pl.BlockSpec((n, 1), lambda i: (0, 0)),
            pl.BlockSpec((1, 2 * d), lambda i: (0, 0)),
            pl.BlockSpec((1, 1), lambda i: (0, 0)),
        ],
        out_specs=pl.BlockSpec((NUM_GRAPHS, 2 * d), lambda i: (0, 0)),
        out_shape=jax.ShapeDtypeStruct((NUM_GRAPHS, 2 * d), jnp.float32),
    )(h, x, batch2d, wg2d, bg2d)


def kernel(x, edge_index, batch, Wl, W_ih, W_hh, b_ih, b_hh, Wg, bg):
    n, d = x.shape
    num_layers = Wl.shape[0]

    src2d = edge_index[0].reshape(-1, 128)
    dst2d = edge_index[1].reshape(-1, 128)
    wiht = W_ih.T
    whht = W_hh.T
    bih2 = b_ih.reshape(1, -1)
    bhh2 = b_hh.reshape(1, -1)
    batch2d = batch.reshape(n, 1)
    wg2d = Wg.reshape(1, -1)
    bg2d = bg.reshape(1, 1)

    h = x
    m = _matmul(x, Wl[0])
    for i in range(num_layers):
        parts = _edge_agg(m, src2d, dst2d)
        if i + 1 < num_layers:
            h, m = _gru(parts, h, wiht, whht, bih2, bhh2, Wl[i + 1])
        else:
            h = _gru(parts, h, wiht, whht, bih2, bhh2)
    return _pool(h, x, batch2d, wg2d, bg2d)
